# baseline (speedup 1.0000x reference)
import functools

import jax
import jax.numpy as jnp
from jax import lax
from jax.experimental import pallas as pl
from jax.experimental.pallas import tpu as pltpu
from jax.experimental.pallas import tpu_sc as plsc

RES = 1024
CH = 8
NLEV = 4
B, HO, WO = 4, 256, 256
NPIX = B * HO * WO
NW = 32
PXW = NPIX // NW
P = 1024
NCHUNK = PXW // P
NJ = P // 128
HWS = tuple((RES >> n) * (RES >> n) for n in range(NLEV))
CK = 512


def _conv_body(p0, p1, p2, p3, f0, f1, f2, f3,
               pin0, pin1, pout0, pout1, sem_i0, sem_i1, sem_o0, sem_o1):
    wid = lax.axis_index("s") * 2 + lax.axis_index("c")
    iota16 = lax.iota(jnp.int32, 16)
    pins = (pin0, pin1)
    pouts = (pout0, pout1)
    sem_is = (sem_i0, sem_i1)
    sem_os = (sem_o0, sem_o1)

    def fire_in(pn, t0, s):
        for c in range(CH):
            pltpu.async_copy(pn.at[c, pl.ds(t0, CK)], pins[s].at[c], sem_is[s])

    def drain_in(pn, s):
        pltpu.make_async_copy(pn.at[pl.ds(0, CH), pl.ds(0, CK)], pins[s],
                              sem_is[s]).wait()

    def compute(s):
        pin = pins[s]
        pout = pouts[s]

        for c in range(CH):
            @plsc.parallel_loop(0, CK, step=16, unroll=4)
            def g_body(off, c=c):
                vec = pin[c, pl.ds(off, 16)]
                plsc.store_scatter(pout, [iota16 + off, jnp.full((16,), c, jnp.int32)], vec)

    def fire_out(fn, t0, s):
        pltpu.async_copy(pouts[s], fn.at[pl.ds(t0, CK), pl.ds(0, CH)],
                         sem_os[s])

    def drain_out(fn, s):
        pltpu.make_async_copy(pouts[s], fn.at[pl.ds(0, CK), pl.ds(0, CH)],
                              sem_os[s]).wait()

    for n, (pn, fn) in enumerate(((p0, f0), (p1, f1), (p2, f2), (p3, f3))):
        span = HWS[n] // NW
        base = wid * span
        nk = span // CK
        if nk == 1:
            fire_in(pn, base, 0)
            drain_in(pn, 0)
            compute(0)
            fire_out(fn, base, 0)
            drain_out(fn, 0)
        else:
            npairs = nk // 2
            fire_in(pn, base, 0)
            fire_in(pn, base + CK, 1)

            def pair_body(k, carry, pn=pn, fn=fn, base=base, nk=nk):
                k2 = k * 2

                drain_in(pn, 0)

                @pl.when(k > 0)
                def _():
                    drain_out(fn, 0)

                compute(0)
                fire_out(fn, base + k2 * CK, 0)

                @pl.when(k2 + 2 < nk)
                def _():
                    fire_in(pn, base + (k2 + 2) * CK, 0)

                drain_in(pn, 1)

                @pl.when(k > 0)
                def _():
                    drain_out(fn, 1)

                compute(1)
                fire_out(fn, base + (k2 + 1) * CK, 1)

                @pl.when(k2 + 3 < nk)
                def _():
                    fire_in(pn, base + (k2 + 3) * CK, 1)
                return carry

            lax.fori_loop(0, npairs, pair_body, 0)
            drain_out(fn, 0)
            drain_out(fn, 1)


def _mip_body(uv_hbm, lev_hbm, t0, t1, t2, t3, out_hbm,
              uv_v, lev_v,
              wa0, wa1, wa2, wa3, wb0, wb1, wb2, wb3,
              ia0, ia1, ia2, ia3, ib0, ib1, ib2, ib3,
              ca0, ca1, ca2, ca3, cb0, cb1, cb2, cb3,
              stage_v, sem_ga, sem_gb, sem_o):
    wid = lax.axis_index("s") * 2 + lax.axis_index("c")
    tabs = (t0, t1, t2, t3)
    iota16 = lax.iota(jnp.int32, 16)
    iota2x = iota16 * 2
    wsets = ((wa0, wa1, wa2, wa3), (wb0, wb1, wb2, wb3))
    isets = ((ia0, ia1, ia2, ia3), (ib0, ib1, ib2, ib3))
    csets = ((ca0, ca1, ca2, ca3), (cb0, cb1, cb2, cb3))
    sems = (sem_ga, sem_gb)

    def prep_and_fire(n):
        w = RES >> n
        tab = tabs[n]
        ws = wsets[n % 2]
        iset = isets[n % 2]
        cs = csets[n % 2]
        sem = sems[n % 2]

        def idx_body(j, carry, tab=tab, iset=iset, cs=cs, sem=sem, n=n, w=w,
                     ws=ws):
            @plsc.parallel_loop(j * 128, j * 128 + 128, step=16, unroll=2)
            def grp_body(off, n=n, w=w, ws=ws, iset=iset):
                uu = plsc.load_gather(uv_v, [iota2x + 2 * off])
                vv = plsc.load_gather(uv_v, [iota2x + (2 * off + 1)])
                ix = uu * jnp.float32(w - 1)
                iy = vv * jnp.float32(w - 1)
                ix0 = ix.astype(jnp.int32)
                iy0 = iy.astype(jnp.int32)
                fx = ix - ix0.astype(jnp.float32)
                fy = iy - iy0.astype(jnp.float32)
                if n < NLEV - 1:
                    lev = lev_v[pl.ds(off, 16)]
                    m = jnp.where(lev <= n, jnp.float32(1.0), jnp.float32(0.0))
                    fym = fy * m
                    my = m - fym
                else:
                    fym = fy
                    my = jnp.float32(1.0) - fy
                gx = jnp.float32(1.0) - fx
                sl = pl.ds(off, 16)
                ws[0][sl] = gx * my
                ws[1][sl] = fx * my
                ws[2][sl] = gx * fym
                ws[3][sl] = fx * fym
                i0 = iy0 * w + ix0
                iset[0][sl] = i0
                iset[1][sl] = i0 + 1
                iset[2][sl] = i0 + w
                iset[3][sl] = i0 + (w + 1)

            ssl = pl.ds(j * 128, 128)
            dsl = pl.ds(j * 128, 128)
            pltpu.async_copy(tab.at[iset[0].at[ssl]], cs[0].at[dsl], sem)
            pltpu.async_copy(tab.at[iset[1].at[ssl]], cs[1].at[dsl], sem)
            pltpu.async_copy(tab.at[iset[2].at[ssl]], cs[2].at[dsl], sem)
            pltpu.async_copy(tab.at[iset[3].at[ssl]], cs[3].at[dsl], sem)
            return carry

        lax.fori_loop(0, NJ, idx_body, 0)

    def drain_gathers(n):
        tab = tabs[n]
        for cb in csets[n % 2]:
            pltpu.make_async_copy(tab.at[pl.ds(0, P)], cb, sems[n % 2]).wait()

    def interp(n):
        ws = wsets[n % 2]
        cs = csets[n % 2]

        @plsc.parallel_loop(0, P, step=16, unroll=8)
        def interp_body(off, n=n, ws=ws, cs=cs):
            a00 = ws[0][pl.ds(off, 16)]
            a01 = ws[1][pl.ds(off, 16)]
            a10 = ws[2][pl.ds(off, 16)]
            a11 = ws[3][pl.ds(off, 16)]
            rows = iota16 + off
            q = lax.shift_right_logical(off, 8)
            o = lax.bitwise_and(off, 255)
            for c in range(CH):
                col = jnp.full((16,), c, jnp.int32)
                v00 = plsc.load_gather(cs[0], [rows, col])
                v01 = plsc.load_gather(cs[1], [rows, col])
                v10 = plsc.load_gather(cs[2], [rows, col])
                v11 = plsc.load_gather(cs[3], [rows, col])
                acc = (v00 * a00 + v01 * a01) + (v10 * a10 + v11 * a11)
                stage_v[n * CH + c, q, pl.ds(o, 16)] = acc

    def drain_out():
        pltpu.make_async_copy(
            stage_v,
            out_hbm.at[0, pl.ds(0, NLEV * CH), pl.ds(0, 4), pl.ds(0, WO)],
            sem_o).wait()

    def chunk_body(t, carry):
        base = wid * PXW + t * P
        pltpu.sync_copy(uv_hbm.at[pl.ds(2 * base, 2 * P)], uv_v)
        pltpu.sync_copy(lev_hbm.at[pl.ds(base, P)], lev_v)

        prep_and_fire(0)
        prep_and_fire(1)
        drain_gathers(0)

        @pl.when(t > 0)
        def _():
            drain_out()

        interp(0)
        for n in range(1, NLEV):
            if n + 1 < NLEV:
                prep_and_fire(n + 1)
            drain_gathers(n)
            interp(n)

        bidx = wid // 8
        r0 = (wid % 8) * 32 + t * 4
        for r in range(NLEV * CH):
            pltpu.async_copy(stage_v.at[r],
                             out_hbm.at[bidx, r, pl.ds(r0, 4), pl.ds(0, WO)],
                             sem_o)
        return carry

    lax.fori_loop(0, NCHUNK, chunk_body, 0)
    drain_out()


@functools.partial(jax.jit, static_argnums=())
def _mip_call(uvf, levf, p0, p1, p2, p3):
    conv = pl.kernel(
        _conv_body,
        out_type=tuple(jax.ShapeDtypeStruct((hw, CH), jnp.float32)
                       for hw in HWS),
        mesh=plsc.VectorSubcoreMesh(core_axis_name="c", subcore_axis_name="s"),
        compiler_params=pltpu.CompilerParams(
            needs_layout_passes=False, use_tc_tiling_on_sc=False),
        scratch_types=(
            [pltpu.VMEM((CH, CK), jnp.float32)] * 2
            + [pltpu.VMEM((CK, CH), jnp.float32)] * 2
            + [pltpu.SemaphoreType.DMA] * 4
        ),
    )
    tabs = conv(p0, p1, p2, p3)

    fn = pl.kernel(
        _mip_body,
        out_type=jax.ShapeDtypeStruct((B, NLEV * CH, HO, WO), jnp.float32),
        mesh=plsc.VectorSubcoreMesh(core_axis_name="c", subcore_axis_name="s"),
        compiler_params=pltpu.CompilerParams(
            needs_layout_passes=False, use_tc_tiling_on_sc=False),
        scratch_types=(
            [pltpu.VMEM((2 * P,), jnp.float32)]
            + [pltpu.VMEM((P,), jnp.int32)]
            + [pltpu.VMEM((P,), jnp.float32)] * 8
            + [pltpu.VMEM((P,), jnp.int32)] * 8
            + [pltpu.VMEM((P, CH), jnp.float32)] * 8
            + [pltpu.VMEM((NLEV * CH, 4, WO), jnp.float32)]
            + [pltpu.SemaphoreType.DMA] * 3
        ),
    )
    return fn(uvf, levf, *tabs)


def kernel(uvs, level, tex0, tex1, tex2, tex3):
    uvf = uvs.reshape(-1)
    levf = level.reshape(-1)
    planes = [t.reshape(CH, -1) for t in (tex0, tex1, tex2, tex3)]
    return _mip_call(uvf, levf, *planes)


# speedup vs baseline: 1.0467x; 1.0467x over previous
import functools

import jax
import jax.numpy as jnp
from jax import lax
from jax.experimental import pallas as pl
from jax.experimental.pallas import tpu as pltpu
from jax.experimental.pallas import tpu_sc as plsc

RES = 1024
CH = 8
NLEV = 4
B, HO, WO = 4, 256, 256
NPIX = B * HO * WO
NW = 32
PXW = NPIX // NW
P = 1024
NCHUNK = PXW // P
NJ = P // 128
HWS = tuple((RES >> n) * (RES >> n) for n in range(NLEV))
CK = 512


def _conv_body(p0, p1, p2, p3, f0, f1, f2, f3,
               pin0, pin1, pout0, pout1, sem_i0, sem_i1, sem_o0, sem_o1):
    wid = lax.axis_index("s") * 2 + lax.axis_index("c")
    iota16 = lax.iota(jnp.int32, 16)
    pins = (pin0, pin1)
    pouts = (pout0, pout1)
    sem_is = (sem_i0, sem_i1)
    sem_os = (sem_o0, sem_o1)

    def fire_in(pn, t0, s):
        for c in range(CH):
            pltpu.async_copy(pn.at[c, pl.ds(t0, CK)], pins[s].at[c], sem_is[s])

    def drain_in(pn, s):
        pltpu.make_async_copy(pn.at[pl.ds(0, CH), pl.ds(0, CK)], pins[s],
                              sem_is[s]).wait()

    def compute(s):
        pin = pins[s]
        pout = pouts[s]

        for c in range(CH):
            @plsc.parallel_loop(0, CK, step=16, unroll=4)
            def g_body(off, c=c):
                vec = pin[c, pl.ds(off, 16)]
                plsc.store_scatter(pout, [iota16 + off, jnp.full((16,), c, jnp.int32)], vec)

    def fire_out(fn, t0, s):
        pltpu.async_copy(pouts[s], fn.at[pl.ds(t0, CK), pl.ds(0, CH)],
                         sem_os[s])

    def drain_out(fn, s):
        pltpu.make_async_copy(pouts[s], fn.at[pl.ds(0, CK), pl.ds(0, CH)],
                              sem_os[s]).wait()

    for n, (pn, fn) in enumerate(((p0, f0), (p1, f1), (p2, f2), (p3, f3))):
        span = HWS[n] // NW
        base = wid * span
        nk = span // CK
        if nk == 1:
            fire_in(pn, base, 0)
            drain_in(pn, 0)
            compute(0)
            fire_out(fn, base, 0)
            drain_out(fn, 0)
        else:
            npairs = nk // 2
            fire_in(pn, base, 0)
            fire_in(pn, base + CK, 1)

            def pair_body(k, carry, pn=pn, fn=fn, base=base, nk=nk):
                k2 = k * 2

                drain_in(pn, 0)

                @pl.when(k > 0)
                def _():
                    drain_out(fn, 0)

                compute(0)
                fire_out(fn, base + k2 * CK, 0)

                @pl.when(k2 + 2 < nk)
                def _():
                    fire_in(pn, base + (k2 + 2) * CK, 0)

                drain_in(pn, 1)

                @pl.when(k > 0)
                def _():
                    drain_out(fn, 1)

                compute(1)
                fire_out(fn, base + (k2 + 1) * CK, 1)

                @pl.when(k2 + 3 < nk)
                def _():
                    fire_in(pn, base + (k2 + 3) * CK, 1)
                return carry

            lax.fori_loop(0, npairs, pair_body, 0)
            drain_out(fn, 0)
            drain_out(fn, 1)


def _mip_body(uv_hbm, lev_hbm, t0, t1, t2, t3, out_hbm,
              uv_v, lev_v,
              wa0, wa1, wa2, wa3, wb0, wb1, wb2, wb3,
              ia0, ia1, ia2, ia3, ib0, ib1, ib2, ib3,
              ca0, ca1, ca2, ca3, cb0, cb1, cb2, cb3,
              stage_v, sem_ga, sem_gb, sem_o):
    wid = lax.axis_index("s") * 2 + lax.axis_index("c")
    tabs = (t0, t1, t2, t3)
    iota16 = lax.iota(jnp.int32, 16)
    iota2x = iota16 * 2
    wsets = ((wa0, wa1, wa2, wa3), (wb0, wb1, wb2, wb3))
    isets = ((ia0, ia1, ia2, ia3), (ib0, ib1, ib2, ib3))
    csets = ((ca0, ca1, ca2, ca3), (cb0, cb1, cb2, cb3))
    sems = (sem_ga, sem_gb)

    def prep_and_fire(n):
        w = RES >> n
        tab = tabs[n]
        ws = wsets[n % 2]
        iset = isets[n % 2]
        cs = csets[n % 2]
        sem = sems[n % 2]

        def idx_body(j, carry, tab=tab, iset=iset, cs=cs, sem=sem, n=n, w=w,
                     ws=ws):
            @plsc.parallel_loop(j * 128, j * 128 + 128, step=16, unroll=2)
            def grp_body(off, n=n, w=w, ws=ws, iset=iset):
                uu = plsc.load_gather(uv_v, [iota2x + 2 * off])
                vv = plsc.load_gather(uv_v, [iota2x + (2 * off + 1)])
                ix = uu * jnp.float32(w - 1)
                iy = vv * jnp.float32(w - 1)
                ix0 = ix.astype(jnp.int32)
                iy0 = iy.astype(jnp.int32)
                fx = ix - ix0.astype(jnp.float32)
                fy = iy - iy0.astype(jnp.float32)
                if n < NLEV - 1:
                    lev = lev_v[pl.ds(off, 16)]
                    m = jnp.where(lev <= n, jnp.float32(1.0), jnp.float32(0.0))
                    fym = fy * m
                    my = m - fym
                else:
                    fym = fy
                    my = jnp.float32(1.0) - fy
                gx = jnp.float32(1.0) - fx
                sl = pl.ds(off, 16)
                ws[0][sl] = gx * my
                ws[1][sl] = fx * my
                ws[2][sl] = gx * fym
                ws[3][sl] = fx * fym
                i0 = iy0 * w + ix0
                iset[0][sl] = i0
                iset[1][sl] = i0 + 1
                iset[2][sl] = i0 + w
                iset[3][sl] = i0 + (w + 1)

            ssl = pl.ds(j * 128, 128)
            dsl = pl.ds(j * 128, 128)
            pltpu.async_copy(tab.at[iset[0].at[ssl]], cs[0].at[dsl], sem)
            pltpu.async_copy(tab.at[iset[1].at[ssl]], cs[1].at[dsl], sem)
            pltpu.async_copy(tab.at[iset[2].at[ssl]], cs[2].at[dsl], sem)
            pltpu.async_copy(tab.at[iset[3].at[ssl]], cs[3].at[dsl], sem)
            return carry

        lax.fori_loop(0, NJ, idx_body, 0)

    def drain_gathers(n):
        tab = tabs[n]
        for cb in csets[n % 2]:
            pltpu.make_async_copy(tab.at[pl.ds(0, P)], cb, sems[n % 2]).wait()

    def interp(n):
        ws = wsets[n % 2]
        cs = csets[n % 2]

        @plsc.parallel_loop(0, P, step=16, unroll=4)
        def interp_body(off, n=n, ws=ws, cs=cs):
            a00 = ws[0][pl.ds(off, 16)]
            a01 = ws[1][pl.ds(off, 16)]
            a10 = ws[2][pl.ds(off, 16)]
            a11 = ws[3][pl.ds(off, 16)]
            rows = iota16 + off
            q = lax.shift_right_logical(off, 8)
            o = lax.bitwise_and(off, 255)
            for c in range(CH):
                col = jnp.full((16,), c, jnp.int32)
                v00 = plsc.load_gather(cs[0], [rows, col])
                v01 = plsc.load_gather(cs[1], [rows, col])
                v10 = plsc.load_gather(cs[2], [rows, col])
                v11 = plsc.load_gather(cs[3], [rows, col])
                acc = (v00 * a00 + v01 * a01) + (v10 * a10 + v11 * a11)
                stage_v[n * CH + c, q, pl.ds(o, 16)] = acc

    def drain_out():
        pltpu.make_async_copy(
            stage_v,
            out_hbm.at[0, pl.ds(0, NLEV * CH), pl.ds(0, 4), pl.ds(0, WO)],
            sem_o).wait()

    def chunk_body(t, carry):
        base = wid * PXW + t * P
        pltpu.sync_copy(uv_hbm.at[pl.ds(2 * base, 2 * P)], uv_v)
        pltpu.sync_copy(lev_hbm.at[pl.ds(base, P)], lev_v)

        prep_and_fire(0)
        prep_and_fire(1)
        drain_gathers(0)

        @pl.when(t > 0)
        def _():
            drain_out()

        interp(0)
        for n in range(1, NLEV):
            if n + 1 < NLEV:
                prep_and_fire(n + 1)
            drain_gathers(n)
            interp(n)

        bidx = wid // 8
        r0 = (wid % 8) * 32 + t * 4
        for r in range(NLEV * CH):
            pltpu.async_copy(stage_v.at[r],
                             out_hbm.at[bidx, r, pl.ds(r0, 4), pl.ds(0, WO)],
                             sem_o)
        return carry

    lax.fori_loop(0, NCHUNK, chunk_body, 0)
    drain_out()


@functools.partial(jax.jit, static_argnums=())
def _mip_call(uvf, levf, p0, p1, p2, p3):
    conv = pl.kernel(
        _conv_body,
        out_type=tuple(jax.ShapeDtypeStruct((hw, CH), jnp.float32)
                       for hw in HWS),
        mesh=plsc.VectorSubcoreMesh(core_axis_name="c", subcore_axis_name="s"),
        compiler_params=pltpu.CompilerParams(
            needs_layout_passes=False, use_tc_tiling_on_sc=False),
        scratch_types=(
            [pltpu.VMEM((CH, CK), jnp.float32)] * 2
            + [pltpu.VMEM((CK, CH), jnp.float32)] * 2
            + [pltpu.SemaphoreType.DMA] * 4
        ),
    )
    tabs = conv(p0, p1, p2, p3)

    fn = pl.kernel(
        _mip_body,
        out_type=jax.ShapeDtypeStruct((B, NLEV * CH, HO, WO), jnp.float32),
        mesh=plsc.VectorSubcoreMesh(core_axis_name="c", subcore_axis_name="s"),
        compiler_params=pltpu.CompilerParams(
            needs_layout_passes=False, use_tc_tiling_on_sc=False),
        scratch_types=(
            [pltpu.VMEM((2 * P,), jnp.float32)]
            + [pltpu.VMEM((P,), jnp.int32)]
            + [pltpu.VMEM((P,), jnp.float32)] * 8
            + [pltpu.VMEM((P,), jnp.int32)] * 8
            + [pltpu.VMEM((P, CH), jnp.float32)] * 8
            + [pltpu.VMEM((NLEV * CH, 4, WO), jnp.float32)]
            + [pltpu.SemaphoreType.DMA] * 3
        ),
    )
    return fn(uvf, levf, *tabs)


def kernel(uvs, level, tex0, tex1, tex2, tex3):
    uvf = uvs.reshape(-1)
    levf = level.reshape(-1)
    planes = [t.reshape(CH, -1) for t in (tex0, tex1, tex2, tex3)]
    return _mip_call(uvf, levf, *planes)


# speedup vs baseline: 1.0813x; 1.0330x over previous
import functools

import jax
import jax.numpy as jnp
from jax import lax
from jax.experimental import pallas as pl
from jax.experimental.pallas import tpu as pltpu
from jax.experimental.pallas import tpu_sc as plsc

RES = 1024
CH = 8
NLEV = 4
B, HO, WO = 4, 256, 256
NPIX = B * HO * WO
NW = 32
PXW = NPIX // NW
P = 1024
NCHUNK = PXW // P
NJ = P // 128
HWS = tuple((RES >> n) * (RES >> n) for n in range(NLEV))
CK = 512


def _conv_body(p0, p1, p2, p3, f0, f1, f2, f3,
               pin0, pin1, pout0, pout1, sem_i0, sem_i1, sem_o0, sem_o1):
    wid = lax.axis_index("s") * 2 + lax.axis_index("c")
    iota16 = lax.iota(jnp.int32, 16)
    pins = (pin0, pin1)
    pouts = (pout0, pout1)
    sem_is = (sem_i0, sem_i1)
    sem_os = (sem_o0, sem_o1)

    def fire_in(pn, t0, s):
        for c in range(CH):
            pltpu.async_copy(pn.at[c, pl.ds(t0, CK)], pins[s].at[c], sem_is[s])

    def drain_in(pn, s):
        pltpu.make_async_copy(pn.at[pl.ds(0, CH), pl.ds(0, CK)], pins[s],
                              sem_is[s]).wait()

    def compute(s):
        pin = pins[s]
        pout = pouts[s]

        for c in range(CH):
            @plsc.parallel_loop(0, CK, step=16, unroll=4)
            def g_body(off, c=c):
                vec = pin[c, pl.ds(off, 16)]
                plsc.store_scatter(pout, [iota16 + off, jnp.full((16,), c, jnp.int32)], vec)

    def fire_out(fn, t0, s):
        pltpu.async_copy(pouts[s], fn.at[pl.ds(t0, CK), pl.ds(0, CH)],
                         sem_os[s])

    def drain_out(fn, s):
        pltpu.make_async_copy(pouts[s], fn.at[pl.ds(0, CK), pl.ds(0, CH)],
                              sem_os[s]).wait()

    for n, (pn, fn) in enumerate(((p0, f0), (p1, f1), (p2, f2), (p3, f3))):
        span = HWS[n] // NW
        base = wid * span
        nk = span // CK
        if nk == 1:
            fire_in(pn, base, 0)
            drain_in(pn, 0)
            compute(0)
            fire_out(fn, base, 0)
            drain_out(fn, 0)
        else:
            npairs = nk // 2
            fire_in(pn, base, 0)
            fire_in(pn, base + CK, 1)

            def pair_body(k, carry, pn=pn, fn=fn, base=base, nk=nk):
                k2 = k * 2

                drain_in(pn, 0)

                @pl.when(k > 0)
                def _():
                    drain_out(fn, 0)

                compute(0)
                fire_out(fn, base + k2 * CK, 0)

                @pl.when(k2 + 2 < nk)
                def _():
                    fire_in(pn, base + (k2 + 2) * CK, 0)

                drain_in(pn, 1)

                @pl.when(k > 0)
                def _():
                    drain_out(fn, 1)

                compute(1)
                fire_out(fn, base + (k2 + 1) * CK, 1)

                @pl.when(k2 + 3 < nk)
                def _():
                    fire_in(pn, base + (k2 + 3) * CK, 1)
                return carry

            lax.fori_loop(0, npairs, pair_body, 0)
            drain_out(fn, 0)
            drain_out(fn, 1)


def _mip_body(uv_hbm, lev_hbm, t0, t1, t2, t3, out_hbm,
              uv_v, lev_v,
              wa0, wa1, wa2, wa3, wb0, wb1, wb2, wb3,
              ia0, ia1, ia2, ia3, ib0, ib1, ib2, ib3,
              ca0, ca1, ca2, ca3, cb0, cb1, cb2, cb3,
              stage_v, sem_ga, sem_gb, sem_o):
    wid = lax.axis_index("s") * 2 + lax.axis_index("c")
    tabs = (t0, t1, t2, t3)
    iota16 = lax.iota(jnp.int32, 16)
    iota2x = iota16 * 2
    wsets = ((wa0, wa1, wa2, wa3), (wb0, wb1, wb2, wb3))
    isets = ((ia0, ia1, ia2, ia3), (ib0, ib1, ib2, ib3))
    csets = ((ca0, ca1, ca2, ca3), (cb0, cb1, cb2, cb3))
    sems = (sem_ga, sem_gb)

    def prep_and_fire(n):
        w = RES >> n
        tab = tabs[n]
        ws = wsets[n % 2]
        iset = isets[n % 2]
        cs = csets[n % 2]
        sem = sems[n % 2]

        def idx_body(j, carry, tab=tab, iset=iset, cs=cs, sem=sem, n=n, w=w,
                     ws=ws):
            @plsc.parallel_loop(j * 128, j * 128 + 128, step=16, unroll=2)
            def grp_body(off, n=n, w=w, ws=ws, iset=iset):
                uu = plsc.load_gather(uv_v, [iota2x + 2 * off])
                vv = plsc.load_gather(uv_v, [iota2x + (2 * off + 1)])
                ix = uu * jnp.float32(w - 1)
                iy = vv * jnp.float32(w - 1)
                ix0 = ix.astype(jnp.int32)
                iy0 = iy.astype(jnp.int32)
                fx = ix - ix0.astype(jnp.float32)
                fy = iy - iy0.astype(jnp.float32)
                if n < NLEV - 1:
                    lev = lev_v[pl.ds(off, 16)]
                    m = jnp.where(lev <= n, jnp.float32(1.0), jnp.float32(0.0))
                    fym = fy * m
                    my = m - fym
                else:
                    fym = fy
                    my = jnp.float32(1.0) - fy
                gx = jnp.float32(1.0) - fx
                sl = pl.ds(off, 16)
                ws[0][sl] = gx * my
                ws[1][sl] = fx * my
                ws[2][sl] = gx * fym
                ws[3][sl] = fx * fym
                i0 = iy0 * w + ix0
                iset[0][sl] = i0
                iset[1][sl] = i0 + 1
                iset[2][sl] = i0 + w
                iset[3][sl] = i0 + (w + 1)

            ssl = pl.ds(j * 128, 128)
            dsl = pl.ds(j * 128, 128)
            pltpu.async_copy(tab.at[iset[0].at[ssl]], cs[0].at[dsl], sem)
            pltpu.async_copy(tab.at[iset[1].at[ssl]], cs[1].at[dsl], sem)
            pltpu.async_copy(tab.at[iset[2].at[ssl]], cs[2].at[dsl], sem)
            pltpu.async_copy(tab.at[iset[3].at[ssl]], cs[3].at[dsl], sem)
            return carry

        lax.fori_loop(0, NJ, idx_body, 0)

    def drain_gathers(n):
        tab = tabs[n]
        for cb in csets[n % 2]:
            pltpu.make_async_copy(tab.at[pl.ds(0, P)], cb, sems[n % 2]).wait()

    def interp(n):
        ws = wsets[n % 2]
        cs = csets[n % 2]

        @plsc.parallel_loop(0, P, step=16, unroll=2)
        def interp_body(off, n=n, ws=ws, cs=cs):
            a00 = ws[0][pl.ds(off, 16)]
            a01 = ws[1][pl.ds(off, 16)]
            a10 = ws[2][pl.ds(off, 16)]
            a11 = ws[3][pl.ds(off, 16)]
            rows = iota16 + off
            q = lax.shift_right_logical(off, 8)
            o = lax.bitwise_and(off, 255)
            for c in range(CH):
                col = jnp.full((16,), c, jnp.int32)
                v00 = plsc.load_gather(cs[0], [rows, col])
                v01 = plsc.load_gather(cs[1], [rows, col])
                v10 = plsc.load_gather(cs[2], [rows, col])
                v11 = plsc.load_gather(cs[3], [rows, col])
                acc = (v00 * a00 + v01 * a01) + (v10 * a10 + v11 * a11)
                stage_v[n * CH + c, q, pl.ds(o, 16)] = acc

    def drain_out():
        pltpu.make_async_copy(
            stage_v,
            out_hbm.at[0, pl.ds(0, NLEV * CH), pl.ds(0, 4), pl.ds(0, WO)],
            sem_o).wait()

    def chunk_body(t, carry):
        base = wid * PXW + t * P
        pltpu.sync_copy(uv_hbm.at[pl.ds(2 * base, 2 * P)], uv_v)
        pltpu.sync_copy(lev_hbm.at[pl.ds(base, P)], lev_v)

        prep_and_fire(0)
        prep_and_fire(1)
        drain_gathers(0)

        @pl.when(t > 0)
        def _():
            drain_out()

        interp(0)
        for n in range(1, NLEV):
            if n + 1 < NLEV:
                prep_and_fire(n + 1)
            drain_gathers(n)
            interp(n)

        bidx = wid // 8
        r0 = (wid % 8) * 32 + t * 4
        for r in range(NLEV * CH):
            pltpu.async_copy(stage_v.at[r],
                             out_hbm.at[bidx, r, pl.ds(r0, 4), pl.ds(0, WO)],
                             sem_o)
        return carry

    lax.fori_loop(0, NCHUNK, chunk_body, 0)
    drain_out()


@functools.partial(jax.jit, static_argnums=())
def _mip_call(uvf, levf, p0, p1, p2, p3):
    conv = pl.kernel(
        _conv_body,
        out_type=tuple(jax.ShapeDtypeStruct((hw, CH), jnp.float32)
                       for hw in HWS),
        mesh=plsc.VectorSubcoreMesh(core_axis_name="c", subcore_axis_name="s"),
        compiler_params=pltpu.CompilerParams(
            needs_layout_passes=False, use_tc_tiling_on_sc=False),
        scratch_types=(
            [pltpu.VMEM((CH, CK), jnp.float32)] * 2
            + [pltpu.VMEM((CK, CH), jnp.float32)] * 2
            + [pltpu.SemaphoreType.DMA] * 4
        ),
    )
    tabs = conv(p0, p1, p2, p3)

    fn = pl.kernel(
        _mip_body,
        out_type=jax.ShapeDtypeStruct((B, NLEV * CH, HO, WO), jnp.float32),
        mesh=plsc.VectorSubcoreMesh(core_axis_name="c", subcore_axis_name="s"),
        compiler_params=pltpu.CompilerParams(
            needs_layout_passes=False, use_tc_tiling_on_sc=False),
        scratch_types=(
            [pltpu.VMEM((2 * P,), jnp.float32)]
            + [pltpu.VMEM((P,), jnp.int32)]
            + [pltpu.VMEM((P,), jnp.float32)] * 8
            + [pltpu.VMEM((P,), jnp.int32)] * 8
            + [pltpu.VMEM((P, CH), jnp.float32)] * 8
            + [pltpu.VMEM((NLEV * CH, 4, WO), jnp.float32)]
            + [pltpu.SemaphoreType.DMA] * 3
        ),
    )
    return fn(uvf, levf, *tabs)


def kernel(uvs, level, tex0, tex1, tex2, tex3):
    uvf = uvs.reshape(-1)
    levf = level.reshape(-1)
    planes = [t.reshape(CH, -1) for t in (tex0, tex1, tex2, tex3)]
    return _mip_call(uvf, levf, *planes)
